# baseline (speedup 1.0000x reference)
import functools

import jax
import jax.numpy as jnp
from jax import lax
from jax.experimental import pallas as pl
from jax.experimental.pallas import tpu as pltpu
from jax.experimental.pallas import tpu_sc as plsc

_NC = 2
_NS = 16
_NW = _NC * _NS
_L = 16
_D = 64
_B = 16384
_BPW = _B // _NW
_CH = 128
_NCH = _BPW // _CH


def _sc_body(i_hbm, j_hbm, te_hbm, ce_hbm, tb_hbm, cb_hbm, out_hbm,
             iv, jv, tv, cv, tbv, cbv, sv, sem):
    wid = lax.axis_index("s") * _NC + lax.axis_index("c")
    base = wid * _BPW
    pltpu.sync_copy(i_hbm.at[wid], iv)
    pltpu.sync_copy(j_hbm.at[wid], jv)
    copies = []
    for k in range(_NCH):
        r = pl.ds(k * _CH, _CH)
        copies.append(pltpu.async_copy(te_hbm.at[iv.at[k]], tv.at[r], sem))
        copies.append(pltpu.async_copy(ce_hbm.at[jv.at[k]], cv.at[r], sem))
        copies.append(pltpu.async_copy(tb_hbm.at[iv.at[k]], tbv.at[r], sem))
        copies.append(pltpu.async_copy(cb_hbm.at[jv.at[k]], cbv.at[r], sem))
    for c in copies:
        c.wait()

    lanes = lax.iota(jnp.int32, _L)

    def group(g, carry):
        o = g * _L
        svec = jnp.zeros((_L,), jnp.float32)
        for r in range(_L):
            row = o + r
            trow = tv.at[row]
            crow = cv.at[row]
            acc = trow[pl.ds(0, _L)] * crow[pl.ds(0, _L)]
            for c in range(1, _D // _L):
                acc = acc + trow[pl.ds(c * _L, _L)] * crow[pl.ds(c * _L, _L)]
            svec = jnp.where(lanes == r, jnp.sum(acc), svec)
        sv[pl.ds(o, _L)] = svec + tbv[pl.ds(o, _L)] + cbv[pl.ds(o, _L)]
        return carry

    lax.fori_loop(0, _BPW // _L, group, 0, unroll=False)
    pltpu.sync_copy(sv, out_hbm.at[pl.ds(base, _BPW)])


@functools.lru_cache(maxsize=1)
def _sc_gather_dot():
    mesh = plsc.VectorSubcoreMesh(core_axis_name="c", subcore_axis_name="s")
    return functools.partial(
        pl.kernel, mesh=mesh,
        compiler_params=pltpu.CompilerParams(
            needs_layout_passes=False, use_tc_tiling_on_sc=False),
        out_type=jax.ShapeDtypeStruct((_B,), jnp.float32),
        scratch_types=[
            pltpu.VMEM((_NCH, _CH), jnp.int32),
            pltpu.VMEM((_NCH, _CH), jnp.int32),
            pltpu.VMEM((_BPW, _D), jnp.float32),
            pltpu.VMEM((_BPW, _D), jnp.float32),
            pltpu.VMEM((_BPW,), jnp.float32),
            pltpu.VMEM((_BPW,), jnp.float32),
            pltpu.VMEM((_BPW,), jnp.float32),
            pltpu.SemaphoreType.DMA,
        ],
    )(_sc_body)


def _tc_cost_body(s_ref, co_ref, out_ref):
    s = s_ref[...]
    co = co_ref[...]
    w = jnp.minimum(1.0, jnp.exp(0.75 * jnp.log(co * (1.0 / 100.0))))
    e = s - jnp.log(co + 1.0)
    out_ref[0, 0] = jnp.sum(w * e * e)


def _tc_cost(s, co):
    out = pl.pallas_call(
        _tc_cost_body,
        out_shape=jax.ShapeDtypeStruct((1, 1), jnp.float32),
        out_specs=pl.BlockSpec(memory_space=pltpu.SMEM),
    )(s.reshape(128, 128), co.reshape(128, 128))
    return out[0, 0]


def kernel(i_ids, j_ids, co_occurs, target_embeddings, context_embeddings,
           target_biases, context_biases):
    i3 = i_ids.astype(jnp.int32).reshape(_NW, _NCH, _CH)
    j3 = j_ids.astype(jnp.int32).reshape(_NW, _NCH, _CH)
    s = _sc_gather_dot()(i3, j3, target_embeddings, context_embeddings,
                         target_biases, context_biases)
    return _tc_cost(s, co_occurs)


# speedup vs baseline: 2.4132x; 2.4132x over previous
import functools

import jax
import jax.numpy as jnp
from jax import lax
from jax.experimental import pallas as pl
from jax.experimental.pallas import tpu as pltpu
from jax.experimental.pallas import tpu_sc as plsc

_NC = 2
_NS = 16
_NW = _NC * _NS
_L = 16
_D = 64
_SL = 8
_B = 16384
_BPW = _B // _NW
_HB = _BPW // 2


def _sc_body(i_hbm, j_hbm, te_hbm, ce_hbm, tb_hbm, cb_hbm, out_hbm,
             iv, jv, tv, cv, tbv, cbv, sv, semt, semc, semb):
    wid = lax.axis_index("s") * _NC + lax.axis_index("c")
    base = wid * _BPW
    pltpu.sync_copy(i_hbm.at[wid], iv)
    pltpu.sync_copy(j_hbm.at[wid], jv)

    bias_copies = []
    for blk in range(4):
        r = pl.ds(blk * 128, 128)
        bias_copies.append(pltpu.async_copy(tb_hbm.at[iv.at[blk]], tbv.at[r], semb))
        bias_copies.append(pltpu.async_copy(cb_hbm.at[jv.at[blk]], cbv.at[r], semb))

    lanes = lax.iota(jnp.int32, _L)

    def half_body(h, carry):
        hbase = h * _HB

        def fire(g, carry):
            rr = hbase + g * _L
            blk = lax.shift_right_logical(rr, 7)
            col = jnp.bitwise_and(rr, 127)
            ivec = iv[blk, pl.ds(col, _L)]
            jvec = jv[blk, pl.ds(col, _L)]
            for r in range(_L):
                ti = ivec[r]
                tj = jvec[r]
                pltpu.async_copy(
                    te_hbm.at[lax.shift_right_logical(ti, 3),
                              jnp.bitwise_and(ti, 7)],
                    tv.at[g * _L + r], semt)
                pltpu.async_copy(
                    ce_hbm.at[lax.shift_right_logical(tj, 3),
                              jnp.bitwise_and(tj, 7)],
                    cv.at[g * _L + r], semc)
            return carry

        lax.fori_loop(0, _HB // _L, fire, 0, unroll=False)

        def drain(r, carry):
            pltpu.make_async_copy(te_hbm.at[0, 0], tv.at[0], semt).wait()
            pltpu.make_async_copy(ce_hbm.at[0, 0], cv.at[0], semc).wait()
            return carry

        lax.fori_loop(0, _HB, drain, 0, unroll=False)

        def group(g, carry):
            o = g * _L
            svec = jnp.zeros((_L,), jnp.float32)
            for r in range(_L):
                acc = tv[o + r, pl.ds(0, _L)] * cv[o + r, pl.ds(0, _L)]
                for c in range(1, _D // _L):
                    acc = acc + (tv[o + r, pl.ds(c * _L, _L)]
                                 * cv[o + r, pl.ds(c * _L, _L)])
                svec = jnp.where(lanes == r, jnp.sum(acc), svec)
            sv[pl.ds(hbase + o, _L)] = svec
            return carry

        lax.fori_loop(0, _HB // _L, group, 0, unroll=False)
        return carry

    lax.fori_loop(0, 2, half_body, 0, unroll=False)

    for c in bias_copies:
        c.wait()
    for g in range(_BPW // _L):
        o = g * _L
        sv[pl.ds(o, _L)] = sv[pl.ds(o, _L)] + tbv[pl.ds(o, _L)] + cbv[pl.ds(o, _L)]
    pltpu.sync_copy(sv, out_hbm.at[pl.ds(base, _BPW)])


@functools.lru_cache(maxsize=1)
def _sc_gather_dot():
    mesh = plsc.VectorSubcoreMesh(core_axis_name="c", subcore_axis_name="s")
    return functools.partial(
        pl.kernel, mesh=mesh,
        compiler_params=pltpu.CompilerParams(needs_layout_passes=False),
        out_type=jax.ShapeDtypeStruct((_B,), jnp.float32),
        scratch_types=[
            pltpu.VMEM((4, 128), jnp.int32),
            pltpu.VMEM((4, 128), jnp.int32),
            pltpu.VMEM((_HB, _D), jnp.float32),
            pltpu.VMEM((_HB, _D), jnp.float32),
            pltpu.VMEM((_BPW,), jnp.float32),
            pltpu.VMEM((_BPW,), jnp.float32),
            pltpu.VMEM((_BPW,), jnp.float32),
            pltpu.SemaphoreType.DMA,
            pltpu.SemaphoreType.DMA,
            pltpu.SemaphoreType.DMA,
        ],
    )(_sc_body)


def _tc_cost_body(s_ref, co_ref, out_ref):
    s = s_ref[...]
    co = co_ref[...]
    w = jnp.minimum(1.0, jnp.exp(0.75 * jnp.log(co * (1.0 / 100.0))))
    e = s - jnp.log(co + 1.0)
    out_ref[0, 0] = jnp.sum(w * e * e)


def _tc_cost(s, co):
    out = pl.pallas_call(
        _tc_cost_body,
        out_shape=jax.ShapeDtypeStruct((1, 1), jnp.float32),
        out_specs=pl.BlockSpec(memory_space=pltpu.SMEM),
    )(s.reshape(128, 128), co.reshape(128, 128))
    return out[0, 0]


def kernel(i_ids, j_ids, co_occurs, target_embeddings, context_embeddings,
           target_biases, context_biases):
    i3 = i_ids.astype(jnp.int32).reshape(_NW, 4, 128)
    j3 = j_ids.astype(jnp.int32).reshape(_NW, 4, 128)
    v = target_embeddings.shape[0]
    te3 = target_embeddings.reshape(v // _SL, _SL, _D)
    ce3 = context_embeddings.reshape(v // _SL, _SL, _D)
    s = _sc_gather_dot()(i3, j3, te3, ce3, target_biases, context_biases)
    return _tc_cost(s, co_occurs)
